# Initial kernel scaffold; baseline (speedup 1.0000x reference)
import jax
import jax.numpy as jnp
from jax.experimental import pallas as pl


def kernel(xyz, feature):
    raise NotImplementedError("write your pallas kernel here")


# speedup vs baseline: 4.3078x; 4.3078x over previous
import functools

import numpy as np
import jax
import jax.numpy as jnp
from jax import lax
from jax.experimental import pallas as pl
from jax.experimental.pallas import tpu as pltpu
from jax.experimental.pallas import tpu_sc as plsc

_RATE = 0.25
_CH = 112
_L = 16


@functools.lru_cache(maxsize=None)
def _sample_rows(B, N, nsample):
    with jax.ensure_compile_time_eval():
        skey = jax.random.key(42)
        idx = np.stack(
            [np.asarray(jax.random.permutation(jax.random.fold_in(skey, b), N)[:nsample])
             for b in range(B)],
            axis=0,
        ).astype(np.int64)
    return idx


@functools.lru_cache(maxsize=None)
def _index_tables(B, N, nsample, nw, ns_pad, tot_pad):
    idx = _sample_rows(B, N, nsample)
    gidx = np.zeros((tot_pad,), np.int32)
    gidx[: B * nsample] = (idx + (np.arange(B, dtype=np.int64) * N)[:, None]).reshape(-1)
    lidx = np.zeros((B, 1, ns_pad), np.int32)
    lidx[:, 0, :nsample] = idx
    return gidx, lidx


def kernel(xyz, feature):
    B, N, DX = xyz.shape
    _, _, DF = feature.shape
    nsample = int(_RATE * N)
    tot = B * nsample

    mesh = plsc.VectorSubcoreMesh(core_axis_name="c", subcore_axis_name="s")
    nc, ns = mesh.num_cores, mesh.num_subcores
    nw = nc * ns

    pw = -(-tot // (nw * _CH)) * _CH
    nch = pw // _CH
    tot_pad = pw * nw

    npl = DX * B
    ns_pad = -(-nsample // _L) * _L
    nstep = ns_pad // _L

    gidx_np, lidx_np = _index_tables(B, N, nsample, nw, ns_pad, tot_pad)
    gidx = jnp.asarray(gidx_np).reshape(nw, nch, _CH)
    lidx = jnp.asarray(lidx_np)

    feat_flat = feature.reshape(B * N, DF)
    planes = xyz.transpose(2, 0, 1).reshape(npl, 1, N)

    @functools.partial(
        pl.kernel,
        out_type=(
            jax.ShapeDtypeStruct((npl, 1, ns_pad), xyz.dtype),
            jax.ShapeDtypeStruct((tot_pad, DF), feature.dtype),
        ),
        mesh=mesh,
        compiler_params=pltpu.CompilerParams(needs_layout_passes=False),
        scratch_types=[
            pltpu.VMEM((nch, _CH), jnp.int32),
            pltpu.VMEM((_CH, DF), jnp.float32),
            pltpu.VMEM((1, N), jnp.float32),
            pltpu.VMEM((1, ns_pad), jnp.int32),
            pltpu.VMEM((1, ns_pad), jnp.float32),
            pltpu.SemaphoreType.DMA,
        ],
    )
    def gather_rows(planes_hbm, feat_hbm, gidx_hbm, lidx_hbm, xout_hbm, fout_hbm,
                    idx_v, fbuf, plane_v, lidx_v, xres_v, fsem):
        wid = lax.axis_index("s") * nc + lax.axis_index("c")

        @pl.when(wid < npl)
        def _xyz():
            b = lax.rem(wid, B)
            pltpu.sync_copy(planes_hbm.at[wid], plane_v)
            pltpu.sync_copy(lidx_hbm.at[b], lidx_v)
            zero16 = jnp.zeros((_L,), jnp.int32)

            def step(j, carry):
                ids = lidx_v[0, pl.ds(j * _L, _L)]
                xres_v[0, pl.ds(j * _L, _L)] = plsc.load_gather(plane_v, [zero16, ids])
                return carry

            lax.fori_loop(0, nstep, step, 0)
            pltpu.sync_copy(xres_v, xout_hbm.at[wid])

        pltpu.sync_copy(gidx_hbm.at[wid], idx_v)
        base = wid * nch * _CH
        for c in range(nch):
            pltpu.async_copy(feat_hbm.at[idx_v.at[c]], fbuf, fsem).wait()
            pltpu.sync_copy(fbuf, fout_hbm.at[pl.ds(base + c * _CH, _CH)])

    xout, fout = gather_rows(planes, feat_flat, gidx, lidx)
    sampled_xyz = xout.reshape(DX, B, ns_pad)[:, :, :nsample].transpose(1, 2, 0)
    sampled_feature = fout[:tot].reshape(B, nsample, DF)
    return sampled_xyz, sampled_feature
